# Initial kernel scaffold; baseline (speedup 1.0000x reference)
import jax
import jax.numpy as jnp
from jax.experimental import pallas as pl


def kernel(x, edge_index, W_in, mu0, Wp0, mu1, Wp1):
    raise NotImplementedError("write your pallas kernel here")


# speedup vs baseline: 7.0675x; 7.0675x over previous
import functools

import jax
import jax.numpy as jnp
from jax import lax
from jax.experimental import pallas as pl
from jax.experimental.pallas import tpu as pltpu
from jax.experimental.pallas import tpu_sc as plsc

SIGMA = 0.5
NC = 2
NS = 16
NW = NC * NS
CH = 128


def _normalize(a):
    return a / (jnp.sqrt(jnp.sum(a * a, axis=-1, keepdims=True)) + 1e-6)


def _embed(a, w):
    an = _normalize(a)
    wn = _normalize(w)
    d2 = (jnp.sum(an * an, axis=-1, keepdims=True)
          - 2.0 * lax.dot_general(an, wn, (((1,), (1,)), ((), ())),
                                  preferred_element_type=jnp.float32)
          + jnp.sum(wn * wn, axis=-1)[None, :])
    return jnp.exp(-d2 / (2.0 * SIGMA * SIGMA))


def _in_embed_tc(x_ref, w_ref, o_ref):
    o_ref[:] = _embed(x_ref[:], w_ref[:])


def _layer_tc(n, aggp_ref, degp_ref, mu_ref, wp_ref, h_ref, o_ref):
    agg = aggp_ref[0, :n, :] + aggp_ref[1, :n, :]
    deg = degp_ref[0, :n, 0:1] + degp_ref[1, :n, 0:1]
    agg = agg / jnp.maximum(deg, 1.0)
    score = (agg - mu_ref[:]) / (SIGMA * SIGMA)
    o_ref[:] = _embed(score, wp_ref[:]) + h_ref[:]


def _make_sc_agg(n_pad, h_dim, nch, with_deg):
    rpt = n_pad // NS
    kz = rpt // CH

    def body(src_hbm, dst_hbm, h_hbm, *refs):
        if with_deg:
            (agg_hbm, deg_hbm, agg_sh, deg_sh,
             src_v, dst_v, rows_v, ones_v, z16_v) = refs
        else:
            agg_hbm, agg_sh, src_v, dst_v, rows_v = refs
        c = lax.axis_index("c")
        s = lax.axis_index("s")
        wid = c * NS + s
        base = s * rpt

        zero16 = jnp.zeros((16,), jnp.float32)
        one16 = jnp.ones((16,), jnp.float32)

        def zrow(r, _):
            for cc in range(h_dim // 16):
                rows_v[r, pl.ds(cc * 16, 16)] = zero16
            if with_deg:
                ones_v[r, :] = one16
                z16_v[r, :] = zero16
            return 0
        lax.fori_loop(0, CH, zrow, 0)

        for k in range(kz):
            pltpu.sync_copy(rows_v, agg_sh.at[pl.ds(base + k * CH, CH)])
            if with_deg:
                pltpu.sync_copy(z16_v, deg_sh.at[pl.ds(base + k * CH, CH)])
        plsc.subcore_barrier()

        pltpu.sync_copy(src_hbm.at[wid], src_v)
        pltpu.sync_copy(dst_hbm.at[wid], dst_v)

        def chunk(j, _):
            pltpu.sync_copy(h_hbm.at[src_v.at[j]], rows_v)
            pltpu.sync_copy(rows_v, agg_sh.at[dst_v.at[j]], add=True)
            if with_deg:
                pltpu.sync_copy(ones_v, deg_sh.at[dst_v.at[j]], add=True)
            return 0
        lax.fori_loop(0, nch, chunk, 0)
        plsc.subcore_barrier()

        for k in range(kz):
            sl = pl.ds(base + k * CH, CH)
            pltpu.sync_copy(agg_sh.at[sl], agg_hbm.at[c, sl])
            if with_deg:
                pltpu.sync_copy(deg_sh.at[sl], deg_hbm.at[c, sl])

    out_type = [jax.ShapeDtypeStruct((NC, n_pad, h_dim), jnp.float32)]
    scratch = [
        pltpu.VMEM_SHARED((n_pad, h_dim), jnp.float32),
    ]
    if with_deg:
        out_type.append(jax.ShapeDtypeStruct((NC, n_pad, 16), jnp.float32))
        scratch.append(pltpu.VMEM_SHARED((n_pad, 16), jnp.float32))
    scratch += [
        pltpu.VMEM((nch, CH), jnp.int32),
        pltpu.VMEM((nch, CH), jnp.int32),
        pltpu.VMEM((CH, h_dim), jnp.float32),
    ]
    if with_deg:
        scratch += [
            pltpu.VMEM((CH, 16), jnp.float32),
            pltpu.VMEM((CH, 16), jnp.float32),
        ]

    mesh = plsc.VectorSubcoreMesh(core_axis_name="c", subcore_axis_name="s")
    return pl.kernel(body, out_type=tuple(out_type), mesh=mesh,
                     scratch_types=tuple(scratch),
                     compiler_params=pltpu.CompilerParams(
                         use_tc_tiling_on_sc=False))


def kernel(x, edge_index, W_in, mu0, Wp0, mu1, Wp1):
    n, _ = x.shape
    h_dim = W_in.shape[0]
    e = edge_index.shape[1]

    n_pad = -(-(n + 1) // (NS * CH)) * (NS * CH)
    ew = -(-e // NW)
    nch = -(-ew // CH)
    ewp = nch * CH

    src = edge_index[0]
    dst = edge_index[1]
    pad_total = NW * ewp - e
    src3 = jnp.pad(src, (0, pad_total)).reshape(NW, nch, CH)
    dst3 = jnp.pad(dst, (0, pad_total), constant_values=n).reshape(NW, nch, CH)

    in_embed = pl.pallas_call(
        _in_embed_tc,
        out_shape=jax.ShapeDtypeStruct((n, h_dim), jnp.float32),
    )
    layer = pl.pallas_call(
        functools.partial(_layer_tc, n),
        out_shape=jax.ShapeDtypeStruct((n, h_dim), jnp.float32),
    )
    sc_agg_deg = _make_sc_agg(n_pad, h_dim, nch, with_deg=True)
    sc_agg = _make_sc_agg(n_pad, h_dim, nch, with_deg=False)

    h0 = in_embed(x, W_in)
    aggp0, degp = sc_agg_deg(src3, dst3, h0)
    h1 = layer(aggp0, degp, mu0, Wp0, h0)
    (aggp1,) = sc_agg(src3, dst3, h1)
    h2 = layer(aggp1, degp, mu1, Wp1, h1)
    return h2
